# Initial kernel scaffold; baseline (speedup 1.0000x reference)
import jax
import jax.numpy as jnp
from jax.experimental import pallas as pl


def kernel(x, W1, g1, b1, W2, g2, b2):
    raise NotImplementedError("write your pallas kernel here")


# speedup vs baseline: 3.0732x; 3.0732x over previous
import functools

import jax
import jax.numpy as jnp
from jax import lax
from jax.experimental import pallas as pl
from jax.experimental.pallas import tpu as pltpu
from jax.experimental.pallas import tpu_sc as plsc

B = 8
C = 3
N = 2048
KNN = 40
BN_TOT = B * N
E = BN_TOT * KNN
TK = 256
TP = 512
NEG_INF = float("-inf")


def _knn_body(xp_ref, xt_ref, idx_ref):
    b = pl.program_id(0)
    xb = xp_ref[0]
    xt = xt_ref[...]
    inner = jnp.dot(xt.astype(jnp.bfloat16), xb.astype(jnp.bfloat16),
                    preferred_element_type=jnp.float32)
    xxr = jnp.sum(xt * xt, axis=1, keepdims=True)
    xxc = jnp.sum(xb * xb, axis=0, keepdims=True)
    d = 2.0 * inner - xxr - xxc
    iota = lax.broadcasted_iota(jnp.int32, (TK, N), 1)
    cols = []
    for _ in range(KNN):
        m = jnp.max(d, axis=1, keepdims=True)
        first = jnp.min(jnp.where(d == m, iota, N), axis=1, keepdims=True)
        cols.append(first)
        d = jnp.where(iota == first, NEG_INF, d)
    idx_ref[...] = jnp.concatenate(cols, axis=1) + b * N


def _knn_call(xp8, xt8):
    nt = N // TK
    return pl.pallas_call(
        _knn_body,
        grid=(B, nt),
        in_specs=[
            pl.BlockSpec((1, 8, N), lambda b, t: (b, 0, 0)),
            pl.BlockSpec((TK, 8), lambda b, t: (b * nt + t, 0)),
        ],
        out_specs=pl.BlockSpec((TK, KNN), lambda b, t: (b * nt + t, 0)),
        out_shape=jax.ShapeDtypeStruct((BN_TOT, KNN), jnp.int32),
    )(xp8, xt8)


def _sc_gather(table, gidx):
    info = plsc.get_sparse_core_info()
    nw = info.num_cores * info.num_subcores
    per_w = E // nw
    chunk = 128
    n_chunks = per_w // chunk
    mesh = plsc.VectorSubcoreMesh(core_axis_name="c", subcore_axis_name="s")

    @functools.partial(
        pl.kernel,
        mesh=mesh,
        out_type=jax.ShapeDtypeStruct((E, 16), jnp.float32),
        compiler_params=pltpu.CompilerParams(use_tc_tiling_on_sc=False),
        scratch_types=[
            pltpu.VMEM((chunk,), jnp.int32),
            pltpu.VMEM((chunk, 16), jnp.float32),
            pltpu.SemaphoreType.DMA,
        ],
    )
    def gather_k(tab_hbm, idx_hbm, out_hbm, idx_v, rows_v, sem):
        wid = lax.axis_index("s") * info.num_cores + lax.axis_index("c")
        base = wid * per_w

        def body(i, carry):
            off = base + i * chunk
            pltpu.sync_copy(idx_hbm.at[pl.ds(off, chunk)], idx_v)
            pltpu.async_copy(tab_hbm.at[idx_v], rows_v, sem).wait()
            pltpu.sync_copy(rows_v, out_hbm.at[pl.ds(off, chunk)])
            return carry

        lax.fori_loop(0, n_chunks, body, 0)

    return gather_k(table, gidx)


def _p1_body(xg_ref, xt_ref, at_ref, ct_ref, s_ref, q_ref):
    i = pl.program_id(0)
    at = at_ref[...]
    bb = jnp.dot(xt_ref[...], ct_ref[...],
                 preferred_element_type=jnp.float32,
                 precision=lax.Precision.HIGHEST)
    s = jnp.zeros((1, 64), jnp.float32)
    q = jnp.zeros((1, 64), jnp.float32)
    for j in range(KNN):
        h1 = jnp.dot(xg_ref[j], at, preferred_element_type=jnp.float32,
                     precision=lax.Precision.HIGHEST) + bb
        s = s + jnp.sum(h1, axis=0, keepdims=True)
        q = q + jnp.sum(h1 * h1, axis=0, keepdims=True)

    @pl.when(i == 0)
    def _():
        s_ref[...] = jnp.zeros_like(s_ref)
        q_ref[...] = jnp.zeros_like(q_ref)

    s_ref[...] += s
    q_ref[...] += q


def _p1_call(xg, xt16, at, ct):
    nt = BN_TOT // TP
    return pl.pallas_call(
        _p1_body,
        grid=(nt,),
        in_specs=[
            pl.BlockSpec((KNN, TP, 16), lambda i: (0, i, 0)),
            pl.BlockSpec((TP, 16), lambda i: (i, 0)),
            pl.BlockSpec((16, 64), lambda i: (0, 0)),
            pl.BlockSpec((16, 64), lambda i: (0, 0)),
        ],
        out_specs=[
            pl.BlockSpec((1, 64), lambda i: (0, 0)),
            pl.BlockSpec((1, 64), lambda i: (0, 0)),
        ],
        out_shape=[
            jax.ShapeDtypeStruct((1, 64), jnp.float32),
            jax.ShapeDtypeStruct((1, 64), jnp.float32),
        ],
    )(xg, xt16, at, ct)


def _p2_body(xg_ref, xt_ref, at_ref, ct_ref, w2_ref, sc1_ref, sh1_ref,
             ymax_ref, s_ref, q_ref):
    i = pl.program_id(0)
    at = at_ref[...]
    w2 = w2_ref[...]
    sc1 = sc1_ref[...]
    sh1 = sh1_ref[...]
    bb = jnp.dot(xt_ref[...], ct_ref[...],
                 preferred_element_type=jnp.float32,
                 precision=lax.Precision.HIGHEST)
    s = jnp.zeros((1, 64), jnp.float32)
    q = jnp.zeros((1, 64), jnp.float32)
    acc = jnp.full((TP, 64), NEG_INF, jnp.float32)
    for j in range(KNN):
        h1 = jnp.dot(xg_ref[j], at, preferred_element_type=jnp.float32,
                     precision=lax.Precision.HIGHEST) + bb
        z = h1 * sc1 + sh1
        r = jnp.where(z >= 0, z, 0.2 * z)
        h2 = jnp.dot(r, w2, preferred_element_type=jnp.float32,
                     precision=lax.Precision.HIGHEST)
        acc = jnp.maximum(acc, h2)
        s = s + jnp.sum(h2, axis=0, keepdims=True)
        q = q + jnp.sum(h2 * h2, axis=0, keepdims=True)
    ymax_ref[...] = acc

    @pl.when(i == 0)
    def _():
        s_ref[...] = jnp.zeros_like(s_ref)
        q_ref[...] = jnp.zeros_like(q_ref)

    s_ref[...] += s
    q_ref[...] += q


def _p2_call(xg, xt16, at, ct, w2t, sc1, sh1):
    nt = BN_TOT // TP
    return pl.pallas_call(
        _p2_body,
        grid=(nt,),
        in_specs=[
            pl.BlockSpec((KNN, TP, 16), lambda i: (0, i, 0)),
            pl.BlockSpec((TP, 16), lambda i: (i, 0)),
            pl.BlockSpec((16, 64), lambda i: (0, 0)),
            pl.BlockSpec((16, 64), lambda i: (0, 0)),
            pl.BlockSpec((64, 64), lambda i: (0, 0)),
            pl.BlockSpec((1, 64), lambda i: (0, 0)),
            pl.BlockSpec((1, 64), lambda i: (0, 0)),
        ],
        out_specs=[
            pl.BlockSpec((TP, 64), lambda i: (i, 0)),
            pl.BlockSpec((1, 64), lambda i: (0, 0)),
            pl.BlockSpec((1, 64), lambda i: (0, 0)),
        ],
        out_shape=[
            jax.ShapeDtypeStruct((BN_TOT, 64), jnp.float32),
            jax.ShapeDtypeStruct((1, 64), jnp.float32),
            jax.ShapeDtypeStruct((1, 64), jnp.float32),
        ],
    )(xg, xt16, at, ct, w2t, sc1, sh1)


def _p3_body(y_ref, sc2_ref, sh2_ref, out_ref):
    z = y_ref[...] * sc2_ref[...] + sh2_ref[...]
    out_ref[...] = jnp.where(z >= 0, z, 0.2 * z)


def _p3_call(ymax, sc2, sh2):
    tile = 2048
    nt = BN_TOT // tile
    return pl.pallas_call(
        _p3_body,
        grid=(nt,),
        in_specs=[
            pl.BlockSpec((tile, 64), lambda i: (i, 0)),
            pl.BlockSpec((1, 64), lambda i: (0, 0)),
            pl.BlockSpec((1, 64), lambda i: (0, 0)),
        ],
        out_specs=pl.BlockSpec((tile, 64), lambda i: (i, 0)),
        out_shape=jax.ShapeDtypeStruct((BN_TOT, 64), jnp.float32),
    )(ymax, sc2, sh2)


def kernel(x, W1, g1, b1, W2, g2, b2):
    xt3 = jnp.transpose(x, (0, 2, 1)).reshape(BN_TOT, C)
    xt8 = jnp.concatenate(
        [xt3, jnp.zeros((BN_TOT, 8 - C), jnp.float32)], axis=1)
    xt16 = jnp.concatenate(
        [xt3, jnp.zeros((BN_TOT, 16 - C), jnp.float32)], axis=1)
    xp8 = jnp.concatenate([x, jnp.zeros((B, 8 - C, N), jnp.float32)], axis=1)

    w1a = W1[:, :C]
    w1c = W1[:, C:] - w1a
    at = jnp.concatenate(
        [w1a.T, jnp.zeros((16 - C, 64), jnp.float32)], axis=0)
    ct = jnp.concatenate(
        [w1c.T, jnp.zeros((16 - C, 64), jnp.float32)], axis=0)
    w2t = W2.T

    idx = _knn_call(xp8, xt8)
    gidx = jnp.transpose(idx, (1, 0)).reshape(E)
    xg = _sc_gather(xt16, gidx).reshape(KNN, BN_TOT, 16)

    s1, q1 = _p1_call(xg, xt16, at, ct)
    cnt = float(E)
    m1 = s1 / cnt
    v1 = q1 / cnt - m1 * m1
    sc1 = (g1.reshape(1, 64) / jnp.sqrt(v1 + 1e-5)).astype(jnp.float32)
    sh1 = b1.reshape(1, 64) - m1 * sc1

    ymax, s2, q2 = _p2_call(xg, xt16, at, ct, w2t, sc1, sh1)
    m2 = s2 / cnt
    v2 = q2 / cnt - m2 * m2
    sc2 = (g2.reshape(1, 64) / jnp.sqrt(v2 + 1e-5)).astype(jnp.float32)
    sh2 = b2.reshape(1, 64) - m2 * sc2

    y = _p3_call(ymax, sc2, sh2)
    return jnp.transpose(y.reshape(B, N, 64), (0, 2, 1))


# speedup vs baseline: 3.8646x; 1.2575x over previous
import functools

import jax
import jax.numpy as jnp
from jax import lax
from jax.experimental import pallas as pl
from jax.experimental.pallas import tpu as pltpu
from jax.experimental.pallas import tpu_sc as plsc

B = 8
C = 3
N = 2048
KNN = 40
BN_TOT = B * N
E = BN_TOT * KNN
TK = 256
TP = 512
NEG_INF = float("-inf")


def _knn_body(xp_ref, xt_ref, idx_ref):
    b = pl.program_id(0)
    xb = xp_ref[0]
    xt = xt_ref[...]
    inner = jnp.dot(xt.astype(jnp.bfloat16), xb.astype(jnp.bfloat16),
                    preferred_element_type=jnp.float32)
    xxr = jnp.sum(xt * xt, axis=1, keepdims=True)
    xxc = jnp.sum(xb * xb, axis=0, keepdims=True)
    d = 2.0 * inner - xxr - xxc
    iota = lax.broadcasted_iota(jnp.int32, (TK, N), 1)
    cols = []
    for _ in range(KNN):
        m = jnp.max(d, axis=1, keepdims=True)
        first = jnp.min(jnp.where(d == m, iota, N), axis=1, keepdims=True)
        cols.append(first)
        d = jnp.where(iota == first, NEG_INF, d)
    idx_ref[...] = jnp.concatenate(cols, axis=1) + b * N


def _knn_call(xp8, xt8):
    nt = N // TK
    return pl.pallas_call(
        _knn_body,
        grid=(B, nt),
        in_specs=[
            pl.BlockSpec((1, 8, N), lambda b, t: (b, 0, 0)),
            pl.BlockSpec((TK, 8), lambda b, t: (b * nt + t, 0)),
        ],
        out_specs=pl.BlockSpec((TK, KNN), lambda b, t: (b * nt + t, 0)),
        out_shape=jax.ShapeDtypeStruct((BN_TOT, KNN), jnp.int32),
    )(xp8, xt8)


def _sc_gather(table, gidx):
    info = plsc.get_sparse_core_info()
    nw = info.num_cores * info.num_subcores
    per_w = E // nw
    chunk = 128
    n_chunks = per_w // chunk
    mesh = plsc.VectorSubcoreMesh(core_axis_name="c", subcore_axis_name="s")

    fires = 8
    super_rows = chunk * fires
    n_super = per_w // super_rows

    @functools.partial(
        pl.kernel,
        mesh=mesh,
        out_type=jax.ShapeDtypeStruct((E, 16), jnp.float32),
        compiler_params=pltpu.CompilerParams(use_tc_tiling_on_sc=False),
        scratch_types=[
            pltpu.VMEM((per_w,), jnp.int32),
            pltpu.VMEM((super_rows, 16), jnp.float32),
            pltpu.SemaphoreType.DMA,
        ],
    )
    def gather_k(tab_hbm, idx_hbm, out_hbm, idx_v, rows_v, sem):
        wid = lax.axis_index("s") * info.num_cores + lax.axis_index("c")
        base = wid * per_w
        pltpu.sync_copy(idx_hbm.at[pl.ds(base, per_w)], idx_v)

        def body(s, carry):
            handles = []
            for b in range(fires):
                off = s * super_rows + b * chunk
                handles.append(pltpu.async_copy(
                    tab_hbm.at[idx_v.at[pl.ds(off, chunk)]],
                    rows_v.at[pl.ds(b * chunk, chunk)], sem))
            for h in handles:
                h.wait()
            pltpu.sync_copy(
                rows_v, out_hbm.at[pl.ds(base + s * super_rows, super_rows)])
            return carry

        lax.fori_loop(0, n_super, body, 0)

    return gather_k(table, gidx)


def _dot3(a, b):
    ah = a.astype(jnp.bfloat16)
    al = (a - ah.astype(jnp.float32)).astype(jnp.bfloat16)
    bh = b.astype(jnp.bfloat16)
    bl = (b - bh.astype(jnp.float32)).astype(jnp.bfloat16)
    hh = jnp.dot(ah, bh, preferred_element_type=jnp.float32)
    hl = jnp.dot(ah, bl, preferred_element_type=jnp.float32)
    lh = jnp.dot(al, bh, preferred_element_type=jnp.float32)
    return hh + (hl + lh)

def _p1_body(xg_ref, xt_ref, at_ref, ct_ref, s_ref, q_ref):
    i = pl.program_id(0)
    at = at_ref[...]
    bb = _dot3(xt_ref[...], ct_ref[...])
    s = jnp.zeros((1, 64), jnp.float32)
    q = jnp.zeros((1, 64), jnp.float32)
    for j in range(KNN):
        h1 = _dot3(xg_ref[j], at) + bb
        s = s + jnp.sum(h1, axis=0, keepdims=True)
        q = q + jnp.sum(h1 * h1, axis=0, keepdims=True)

    @pl.when(i == 0)
    def _():
        s_ref[...] = jnp.zeros_like(s_ref)
        q_ref[...] = jnp.zeros_like(q_ref)

    s_ref[...] += s
    q_ref[...] += q


def _p1_call(xg, xt16, at, ct):
    nt = BN_TOT // TP
    return pl.pallas_call(
        _p1_body,
        grid=(nt,),
        in_specs=[
            pl.BlockSpec((KNN, TP, 16), lambda i: (0, i, 0)),
            pl.BlockSpec((TP, 16), lambda i: (i, 0)),
            pl.BlockSpec((16, 64), lambda i: (0, 0)),
            pl.BlockSpec((16, 64), lambda i: (0, 0)),
        ],
        out_specs=[
            pl.BlockSpec((1, 64), lambda i: (0, 0)),
            pl.BlockSpec((1, 64), lambda i: (0, 0)),
        ],
        out_shape=[
            jax.ShapeDtypeStruct((1, 64), jnp.float32),
            jax.ShapeDtypeStruct((1, 64), jnp.float32),
        ],
    )(xg, xt16, at, ct)


def _p2_body(xg_ref, xt_ref, at_ref, ct_ref, w2_ref, sc1_ref, sh1_ref,
             ymax_ref, s_ref, q_ref):
    i = pl.program_id(0)
    at = at_ref[...]
    w2 = w2_ref[...]
    sc1 = sc1_ref[...]
    sh1 = sh1_ref[...]
    bb = _dot3(xt_ref[...], ct_ref[...])
    s = jnp.zeros((1, 64), jnp.float32)
    q = jnp.zeros((1, 64), jnp.float32)
    acc = jnp.full((TP, 64), NEG_INF, jnp.float32)
    for j in range(KNN):
        h1 = _dot3(xg_ref[j], at) + bb
        z = h1 * sc1 + sh1
        r = jnp.where(z >= 0, z, 0.2 * z)
        h2 = _dot3(r, w2)
        acc = jnp.maximum(acc, h2)
        s = s + jnp.sum(h2, axis=0, keepdims=True)
        q = q + jnp.sum(h2 * h2, axis=0, keepdims=True)
    ymax_ref[...] = acc

    @pl.when(i == 0)
    def _():
        s_ref[...] = jnp.zeros_like(s_ref)
        q_ref[...] = jnp.zeros_like(q_ref)

    s_ref[...] += s
    q_ref[...] += q


def _p2_call(xg, xt16, at, ct, w2t, sc1, sh1):
    nt = BN_TOT // TP
    return pl.pallas_call(
        _p2_body,
        grid=(nt,),
        in_specs=[
            pl.BlockSpec((KNN, TP, 16), lambda i: (0, i, 0)),
            pl.BlockSpec((TP, 16), lambda i: (i, 0)),
            pl.BlockSpec((16, 64), lambda i: (0, 0)),
            pl.BlockSpec((16, 64), lambda i: (0, 0)),
            pl.BlockSpec((64, 64), lambda i: (0, 0)),
            pl.BlockSpec((1, 64), lambda i: (0, 0)),
            pl.BlockSpec((1, 64), lambda i: (0, 0)),
        ],
        out_specs=[
            pl.BlockSpec((TP, 64), lambda i: (i, 0)),
            pl.BlockSpec((1, 64), lambda i: (0, 0)),
            pl.BlockSpec((1, 64), lambda i: (0, 0)),
        ],
        out_shape=[
            jax.ShapeDtypeStruct((BN_TOT, 64), jnp.float32),
            jax.ShapeDtypeStruct((1, 64), jnp.float32),
            jax.ShapeDtypeStruct((1, 64), jnp.float32),
        ],
    )(xg, xt16, at, ct, w2t, sc1, sh1)


def _p3_body(y_ref, sc2_ref, sh2_ref, out_ref):
    z = y_ref[...] * sc2_ref[...] + sh2_ref[...]
    z = jnp.where(z >= 0, z, 0.2 * z)
    r = lax.broadcasted_iota(jnp.int32, (64, 64), 0)
    c = lax.broadcasted_iota(jnp.int32, (64, 64), 1)
    eye = jnp.where(r == c, 1.0, 0.0).astype(jnp.float32)
    out_ref[0] = lax.dot_general(
        eye, z, (((1,), (1,)), ((), ())),
        preferred_element_type=jnp.float32,
        precision=lax.Precision.HIGHEST)


def _p3_call(ymax, sc2, sh2):
    return pl.pallas_call(
        _p3_body,
        grid=(B,),
        in_specs=[
            pl.BlockSpec((N, 64), lambda i: (i, 0)),
            pl.BlockSpec((1, 64), lambda i: (0, 0)),
            pl.BlockSpec((1, 64), lambda i: (0, 0)),
        ],
        out_specs=pl.BlockSpec((1, 64, N), lambda i: (i, 0, 0)),
        out_shape=jax.ShapeDtypeStruct((B, 64, N), jnp.float32),
    )(ymax, sc2, sh2)


def kernel(x, W1, g1, b1, W2, g2, b2):
    xt3 = jnp.transpose(x, (0, 2, 1)).reshape(BN_TOT, C)
    xt8 = jnp.concatenate(
        [xt3, jnp.zeros((BN_TOT, 8 - C), jnp.float32)], axis=1)
    xt16 = jnp.concatenate(
        [xt3, jnp.zeros((BN_TOT, 16 - C), jnp.float32)], axis=1)
    xp8 = jnp.concatenate([x, jnp.zeros((B, 8 - C, N), jnp.float32)], axis=1)

    w1a = W1[:, :C]
    w1c = W1[:, C:] - w1a
    at = jnp.concatenate(
        [w1a.T, jnp.zeros((16 - C, 64), jnp.float32)], axis=0)
    ct = jnp.concatenate(
        [w1c.T, jnp.zeros((16 - C, 64), jnp.float32)], axis=0)
    w2t = W2.T

    idx = _knn_call(xp8, xt8)
    gidx = jnp.transpose(idx, (1, 0)).reshape(E)
    xg = _sc_gather(xt16, gidx).reshape(KNN, BN_TOT, 16)

    s1, q1 = _p1_call(xg, xt16, at, ct)
    cnt = float(E)
    m1 = s1 / cnt
    v1 = q1 / cnt - m1 * m1
    sc1 = (g1.reshape(1, 64) / jnp.sqrt(v1 + 1e-5)).astype(jnp.float32)
    sh1 = b1.reshape(1, 64) - m1 * sc1

    ymax, s2, q2 = _p2_call(xg, xt16, at, ct, w2t, sc1, sh1)
    m2 = s2 / cnt
    v2 = q2 / cnt - m2 * m2
    sc2 = (g2.reshape(1, 64) / jnp.sqrt(v2 + 1e-5)).astype(jnp.float32)
    sh2 = b2.reshape(1, 64) - m2 * sc2

    return _p3_call(ymax, sc2, sh2)


# speedup vs baseline: 4.7346x; 1.2251x over previous
import functools

import jax
import jax.numpy as jnp
from jax import lax
from jax.experimental import pallas as pl
from jax.experimental.pallas import tpu as pltpu
from jax.experimental.pallas import tpu_sc as plsc

B = 8
C = 3
N = 2048
KNN = 40
BN_TOT = B * N
E = BN_TOT * KNN
TK = 256
TP = 512
NEG_INF = float("-inf")


def _knn_body(xp_ref, xt_ref, idx_ref):
    b = pl.program_id(0)
    xb = xp_ref[0]
    xt = xt_ref[...]
    inner = jnp.dot(xt.astype(jnp.bfloat16), xb.astype(jnp.bfloat16),
                    preferred_element_type=jnp.float32)
    xxr = jnp.sum(xt * xt, axis=1, keepdims=True)
    xxc = jnp.sum(xb * xb, axis=0, keepdims=True)
    d = 2.0 * inner - xxr - xxc
    iota = lax.broadcasted_iota(jnp.int32, (TK, N), 1).astype(jnp.float32)
    cols = []
    for _ in range(KNN):
        m = jnp.max(d, axis=1, keepdims=True)
        first = jnp.min(jnp.where(d == m, iota, float(N)), axis=1,
                        keepdims=True)
        cols.append(first)
        d = jnp.where(iota == first, NEG_INF, d)
    idx_f = jnp.concatenate(cols, axis=1)
    r = lax.broadcasted_iota(jnp.int32, (KNN, KNN), 0)
    c = lax.broadcasted_iota(jnp.int32, (KNN, KNN), 1)
    eye = jnp.where(r == c, 1.0, 0.0).astype(jnp.float32)
    idx_t = lax.dot_general(eye, idx_f, (((1,), (1,)), ((), ())),
                            preferred_element_type=jnp.float32,
                            precision=lax.Precision.HIGHEST)
    idx_ref[...] = idx_t.astype(jnp.int32) + b * N


def _knn_call(xp8, xt8):
    nt = N // TK
    return pl.pallas_call(
        _knn_body,
        grid=(B, nt),
        in_specs=[
            pl.BlockSpec((1, 8, N), lambda b, t: (b, 0, 0)),
            pl.BlockSpec((TK, 8), lambda b, t: (b * nt + t, 0)),
        ],
        out_specs=pl.BlockSpec((KNN, TK), lambda b, t: (0, b * nt + t)),
        out_shape=jax.ShapeDtypeStruct((KNN, BN_TOT), jnp.int32),
    )(xp8, xt8)


def _sc_gather(table, gidx):
    info = plsc.get_sparse_core_info()
    nw = info.num_cores * info.num_subcores
    parts = 4
    part_rows = BN_TOT // parts
    items_per_w = (KNN * parts) // nw
    chunk = 128
    fires = 8
    super_rows = chunk * fires
    n_super = part_rows // super_rows
    mesh = plsc.VectorSubcoreMesh(core_axis_name="c", subcore_axis_name="s")

    @functools.partial(
        pl.kernel,
        mesh=mesh,
        out_type=jax.ShapeDtypeStruct((KNN, BN_TOT, 16), jnp.float32),
        compiler_params=pltpu.CompilerParams(use_tc_tiling_on_sc=False),
        scratch_types=[
            pltpu.VMEM((part_rows,), jnp.int32),
            pltpu.VMEM((super_rows, 16), jnp.float32),
            pltpu.SemaphoreType.DMA,
        ],
    )
    def gather_k(tab_hbm, idx_hbm, out_hbm, idx_v, rows_v, sem):
        wid = lax.axis_index("s") * info.num_cores + lax.axis_index("c")

        def item_body(it, carry):
            item = wid * items_per_w + it
            k = item // parts
            off = (item % parts) * part_rows
            pltpu.sync_copy(idx_hbm.at[k, pl.ds(off, part_rows)], idx_v)

            def body(s, carry2):
                handles = []
                for f in range(fires):
                    o = s * super_rows + f * chunk
                    handles.append(pltpu.async_copy(
                        tab_hbm.at[idx_v.at[pl.ds(o, chunk)]],
                        rows_v.at[pl.ds(f * chunk, chunk)], sem))
                for h in handles:
                    h.wait()
                pltpu.sync_copy(
                    rows_v,
                    out_hbm.at[k, pl.ds(off + s * super_rows, super_rows)])
                return carry2

            lax.fori_loop(0, n_super, body, 0)
            return carry

        lax.fori_loop(0, items_per_w, item_body, 0)

    return gather_k(table, gidx)


def _dot3(a, b):
    ah = a.astype(jnp.bfloat16)
    al = (a - ah.astype(jnp.float32)).astype(jnp.bfloat16)
    bh = b.astype(jnp.bfloat16)
    bl = (b - bh.astype(jnp.float32)).astype(jnp.bfloat16)
    hh = jnp.dot(ah, bh, preferred_element_type=jnp.float32)
    hl = jnp.dot(ah, bl, preferred_element_type=jnp.float32)
    lh = jnp.dot(al, bh, preferred_element_type=jnp.float32)
    return hh + (hl + lh)

def _p1_body(xg_ref, xt_ref, at_ref, ct_ref, s_ref, q_ref):
    i = pl.program_id(0)
    at = at_ref[...]
    bb = _dot3(xt_ref[...], ct_ref[...])
    s = jnp.zeros((1, 64), jnp.float32)
    q = jnp.zeros((1, 64), jnp.float32)
    for j in range(KNN):
        h1 = _dot3(xg_ref[j], at) + bb
        s = s + jnp.sum(h1, axis=0, keepdims=True)
        q = q + jnp.sum(h1 * h1, axis=0, keepdims=True)

    @pl.when(i == 0)
    def _():
        s_ref[...] = jnp.zeros_like(s_ref)
        q_ref[...] = jnp.zeros_like(q_ref)

    s_ref[...] += s
    q_ref[...] += q


def _p1_call(xg, xt16, at, ct):
    nt = BN_TOT // TP
    return pl.pallas_call(
        _p1_body,
        grid=(nt,),
        in_specs=[
            pl.BlockSpec((KNN, TP, 16), lambda i: (0, i, 0)),
            pl.BlockSpec((TP, 16), lambda i: (i, 0)),
            pl.BlockSpec((16, 64), lambda i: (0, 0)),
            pl.BlockSpec((16, 64), lambda i: (0, 0)),
        ],
        out_specs=[
            pl.BlockSpec((1, 64), lambda i: (0, 0)),
            pl.BlockSpec((1, 64), lambda i: (0, 0)),
        ],
        out_shape=[
            jax.ShapeDtypeStruct((1, 64), jnp.float32),
            jax.ShapeDtypeStruct((1, 64), jnp.float32),
        ],
    )(xg, xt16, at, ct)


def _p2_body(xg_ref, xt_ref, at_ref, ct_ref, w2_ref, sc1_ref, sh1_ref,
             ymax_ref, s_ref, q_ref):
    i = pl.program_id(0)
    at = at_ref[...]
    w2 = w2_ref[...]
    sc1 = sc1_ref[...]
    sh1 = sh1_ref[...]
    bb = _dot3(xt_ref[...], ct_ref[...])
    s = jnp.zeros((1, 64), jnp.float32)
    q = jnp.zeros((1, 64), jnp.float32)
    acc = jnp.full((TP, 64), NEG_INF, jnp.float32)
    for j in range(KNN):
        h1 = _dot3(xg_ref[j], at) + bb
        z = h1 * sc1 + sh1
        r = jnp.where(z >= 0, z, 0.2 * z)
        h2 = _dot3(r, w2)
        acc = jnp.maximum(acc, h2)
        s = s + jnp.sum(h2, axis=0, keepdims=True)
        q = q + jnp.sum(h2 * h2, axis=0, keepdims=True)
    ymax_ref[...] = acc

    @pl.when(i == 0)
    def _():
        s_ref[...] = jnp.zeros_like(s_ref)
        q_ref[...] = jnp.zeros_like(q_ref)

    s_ref[...] += s
    q_ref[...] += q


def _p2_call(xg, xt16, at, ct, w2t, sc1, sh1):
    nt = BN_TOT // TP
    return pl.pallas_call(
        _p2_body,
        grid=(nt,),
        in_specs=[
            pl.BlockSpec((KNN, TP, 16), lambda i: (0, i, 0)),
            pl.BlockSpec((TP, 16), lambda i: (i, 0)),
            pl.BlockSpec((16, 64), lambda i: (0, 0)),
            pl.BlockSpec((16, 64), lambda i: (0, 0)),
            pl.BlockSpec((64, 64), lambda i: (0, 0)),
            pl.BlockSpec((1, 64), lambda i: (0, 0)),
            pl.BlockSpec((1, 64), lambda i: (0, 0)),
        ],
        out_specs=[
            pl.BlockSpec((TP, 64), lambda i: (i, 0)),
            pl.BlockSpec((1, 64), lambda i: (0, 0)),
            pl.BlockSpec((1, 64), lambda i: (0, 0)),
        ],
        out_shape=[
            jax.ShapeDtypeStruct((BN_TOT, 64), jnp.float32),
            jax.ShapeDtypeStruct((1, 64), jnp.float32),
            jax.ShapeDtypeStruct((1, 64), jnp.float32),
        ],
    )(xg, xt16, at, ct, w2t, sc1, sh1)


def _p3_body(y_ref, sc2_ref, sh2_ref, out_ref):
    z = y_ref[...] * sc2_ref[...] + sh2_ref[...]
    z = jnp.where(z >= 0, z, 0.2 * z)
    r = lax.broadcasted_iota(jnp.int32, (64, 64), 0)
    c = lax.broadcasted_iota(jnp.int32, (64, 64), 1)
    eye = jnp.where(r == c, 1.0, 0.0).astype(jnp.float32)
    out_ref[0] = lax.dot_general(
        eye, z, (((1,), (1,)), ((), ())),
        preferred_element_type=jnp.float32,
        precision=lax.Precision.HIGHEST)


def _p3_call(ymax, sc2, sh2):
    return pl.pallas_call(
        _p3_body,
        grid=(B,),
        in_specs=[
            pl.BlockSpec((N, 64), lambda i: (i, 0)),
            pl.BlockSpec((1, 64), lambda i: (0, 0)),
            pl.BlockSpec((1, 64), lambda i: (0, 0)),
        ],
        out_specs=pl.BlockSpec((1, 64, N), lambda i: (i, 0, 0)),
        out_shape=jax.ShapeDtypeStruct((B, 64, N), jnp.float32),
    )(ymax, sc2, sh2)


def kernel(x, W1, g1, b1, W2, g2, b2):
    xt3 = jnp.transpose(x, (0, 2, 1)).reshape(BN_TOT, C)
    xt8 = jnp.concatenate(
        [xt3, jnp.zeros((BN_TOT, 8 - C), jnp.float32)], axis=1)
    xt16 = jnp.concatenate(
        [xt3, jnp.zeros((BN_TOT, 16 - C), jnp.float32)], axis=1)
    xp8 = jnp.concatenate([x, jnp.zeros((B, 8 - C, N), jnp.float32)], axis=1)

    w1a = W1[:, :C]
    w1c = W1[:, C:] - w1a
    at = jnp.concatenate(
        [w1a.T, jnp.zeros((16 - C, 64), jnp.float32)], axis=0)
    ct = jnp.concatenate(
        [w1c.T, jnp.zeros((16 - C, 64), jnp.float32)], axis=0)
    w2t = W2.T

    gidx = _knn_call(xp8, xt8)
    xg = _sc_gather(xt16, gidx)

    s1, q1 = _p1_call(xg, xt16, at, ct)
    cnt = float(E)
    m1 = s1 / cnt
    v1 = q1 / cnt - m1 * m1
    sc1 = (g1.reshape(1, 64) / jnp.sqrt(v1 + 1e-5)).astype(jnp.float32)
    sh1 = b1.reshape(1, 64) - m1 * sc1

    ymax, s2, q2 = _p2_call(xg, xt16, at, ct, w2t, sc1, sh1)
    m2 = s2 / cnt
    v2 = q2 / cnt - m2 * m2
    sc2 = (g2.reshape(1, 64) / jnp.sqrt(v2 + 1e-5)).astype(jnp.float32)
    sh2 = b2.reshape(1, 64) - m2 * sc2

    return _p3_call(ymax, sc2, sh2)
